# baseline (speedup 1.0000x reference)
import functools

import jax
import jax.numpy as jnp
from jax import lax
from jax.experimental import pallas as pl
from jax.experimental.pallas import tpu as pltpu
from jax.experimental.pallas import tpu_sc as plsc

_BATCH = 4
_NPIX_FINE = 786432
_NPIX_COARSE = _NPIX_FINE // 4
_C = 16
_L = 16

_NC = 2
_NS = 16
_NW = _NC * _NS
_W_PER_B = _NW // _BATCH

_P_PER_W = _NPIX_FINE // _W_PER_B
_F = 2048
_Q = _F // 4
_NSTEPS = _P_PER_W // _F
_NBUF = 2
_NOUTER = _NSTEPS // _NBUF

_DNUMS = lax.GatherDimensionNumbers(
    offset_dims=(), collapsed_slice_dims=(0,), start_index_map=(0,)
)


def _dg(v, idx):
    return lax.gather(
        v, idx[:, None], _DNUMS, (1,),
        mode=lax.GatherScatterMode.PROMISE_IN_BOUNDS,
    )


def _pool_body(x_hbm, out_hbm, in0, in1, ou0, ou1, si0, si1, so0, so1):
    wid = lax.axis_index("s") * _NC + lax.axis_index("c")
    bi = lax.div(wid, _W_PER_B)
    rng = lax.rem(wid, _W_PER_B)
    p_base = rng * _P_PER_W
    q_base = rng * (_P_PER_W // 4)

    ins, outs = (in0, in1), (ou0, ou1)
    isems, osems = (si0, si1), (so0, so1)

    def in_copy(s, k):
        src = x_hbm.at[bi, :, pl.ds(p_base + s * _F, _F)]
        return pltpu.make_async_copy(src, ins[k], isems[k])

    def out_copy(s, k):
        dst = out_hbm.at[bi, :, pl.ds(q_base + s * _Q, _Q)]
        return pltpu.make_async_copy(outs[k], dst, osems[k])

    for k in range(_NBUF):
        in_copy(k, k).start()

    iot = lax.iota(jnp.int32, _L)
    xor1 = iot ^ 1
    xor2 = iot ^ 2
    trp = (iot & 3) * 4 + (iot >> 2)
    m4a = (iot & 3) == 0
    m4b = (iot & 3) == 1
    m4c = (iot & 3) == 2

    def outer(t, _):
        for k in range(_NBUF):
            s = t * _NBUF + k
            in_copy(s, k).wait()

            @plsc.parallel_loop(0, _Q // _L, unroll=8)
            def vbody(v, k=k):
                base = 64 * v
                for c in range(_C):
                    qs = []
                    for i in range(4):
                        vi = ins[k][c, pl.ds(base + _L * i, _L)]
                        t1 = vi + _dg(vi, xor1)
                        qs.append(t1 + _dg(t1, xor2))
                    s2 = jnp.where(
                        m4a, qs[0],
                        jnp.where(m4b, qs[1], jnp.where(m4c, qs[2], qs[3])),
                    )
                    outs[k][c, pl.ds(v * _L, _L)] = _dg(s2, trp) * 0.25

            @pl.when(s >= _NBUF)
            def _(k=k, s=s):
                out_copy(s - _NBUF, k).wait()

            out_copy(s, k).start()

            @pl.when(s + _NBUF < _NSTEPS)
            def _(k=k, s=s):
                in_copy(s + _NBUF, k).start()

        return 0

    lax.fori_loop(0, _NOUTER, outer, 0)

    for k in range(_NBUF):
        out_copy(_NSTEPS - _NBUF + k, k).wait()


@functools.partial(
    pl.kernel,
    out_type=jax.ShapeDtypeStruct((_BATCH, _C, _NPIX_COARSE), jnp.float32),
    mesh=plsc.VectorSubcoreMesh(core_axis_name="c", subcore_axis_name="s"),
    compiler_params=pltpu.CompilerParams(use_tc_tiling_on_sc=True),
    scratch_types=[
        pltpu.VMEM((_C, _F), jnp.float32),
        pltpu.VMEM((_C, _F), jnp.float32),
        pltpu.VMEM((_C, _Q), jnp.float32),
        pltpu.VMEM((_C, _Q), jnp.float32),
        pltpu.SemaphoreType.DMA,
        pltpu.SemaphoreType.DMA,
        pltpu.SemaphoreType.DMA,
        pltpu.SemaphoreType.DMA,
    ],
)
def _pool(x_hbm, out_hbm, in0, in1, ou0, ou1, si0, si1, so0, so1):
    _pool_body(x_hbm, out_hbm, in0, in1, ou0, ou1, si0, si1, so0, so1)


def kernel(x, groups):
    del groups
    xt = jnp.transpose(x, (0, 2, 1))
    out_t = _pool(xt)
    return jnp.transpose(out_t, (0, 2, 1))


# speedup vs baseline: 1.4235x; 1.4235x over previous
import functools

import jax
import jax.numpy as jnp
from jax import lax
from jax.experimental import pallas as pl
from jax.experimental.pallas import tpu as pltpu
from jax.experimental.pallas import tpu_sc as plsc

_BATCH = 4
_NPIX_FINE = 786432
_NPIX_COARSE = _NPIX_FINE // 4
_C = 16
_L = 16

_NC = 2
_NS = 16
_NW = _NC * _NS
_W_PER_B = _NW // _BATCH

_P_PER_W = _NPIX_FINE // _W_PER_B
_F = 2048
_Q = _F // 4
_NSTEPS = _P_PER_W // _F
_NBUF = 3
_NOUTER = _NSTEPS // _NBUF

_DNUMS = lax.GatherDimensionNumbers(
    offset_dims=(), collapsed_slice_dims=(0,), start_index_map=(0,)
)


def _dg(v, idx):
    return lax.gather(
        v, idx[:, None], _DNUMS, (1,),
        mode=lax.GatherScatterMode.PROMISE_IN_BOUNDS,
    )


def _pool_body(x_hbm, out_hbm, in0, in1, in2, ou0, ou1, ou2,
               si0, si1, si2, so0, so1, so2):
    wid = lax.axis_index("s") * _NC + lax.axis_index("c")
    bi = lax.div(wid, _W_PER_B)
    rng = lax.rem(wid, _W_PER_B)
    p_base = rng * _P_PER_W
    q_base = rng * (_P_PER_W // 4)

    ins, outs = (in0, in1, in2), (ou0, ou1, ou2)
    isems, osems = (si0, si1, si2), (so0, so1, so2)

    def in_copy(s, k):
        src = x_hbm.at[bi, :, pl.ds(p_base + s * _F, _F)]
        return pltpu.make_async_copy(src, ins[k], isems[k])

    def out_copy(s, k):
        dst = out_hbm.at[bi, :, pl.ds(q_base + s * _Q, _Q)]
        return pltpu.make_async_copy(outs[k], dst, osems[k])

    for k in range(_NBUF):
        in_copy(k, k).start()

    iot = lax.iota(jnp.int32, _L)
    xor1 = iot ^ 1
    xor2 = iot ^ 2
    trp = (iot & 3) * 4 + (iot >> 2)
    m4a = (iot & 3) == 0
    m4b = (iot & 3) == 1
    m4c = (iot & 3) == 2

    def outer(t, _):
        for k in range(_NBUF):
            s = t * _NBUF + k
            in_copy(s, k).wait()

            @plsc.parallel_loop(0, _Q // _L, unroll=4)
            def vbody(v, k=k):
                base = 64 * v
                for c in range(_C):
                    qs = []
                    for i in range(4):
                        vi = ins[k][c, pl.ds(base + _L * i, _L)]
                        t1 = vi + _dg(vi, xor1)
                        qs.append(t1 + _dg(t1, xor2))
                    s2 = jnp.where(
                        m4a, qs[0],
                        jnp.where(m4b, qs[1], jnp.where(m4c, qs[2], qs[3])),
                    )
                    outs[k][c, pl.ds(v * _L, _L)] = _dg(s2, trp) * 0.25

            @pl.when(s >= _NBUF)
            def _(k=k, s=s):
                out_copy(s - _NBUF, k).wait()

            out_copy(s, k).start()

            @pl.when(s + _NBUF < _NSTEPS)
            def _(k=k, s=s):
                in_copy(s + _NBUF, k).start()

        return 0

    lax.fori_loop(0, _NOUTER, outer, 0)

    for k in range(_NBUF):
        out_copy(_NSTEPS - _NBUF + k, k).wait()


@functools.partial(
    pl.kernel,
    out_type=jax.ShapeDtypeStruct((_BATCH, _C, _NPIX_COARSE), jnp.float32),
    mesh=plsc.VectorSubcoreMesh(core_axis_name="c", subcore_axis_name="s"),
    compiler_params=pltpu.CompilerParams(use_tc_tiling_on_sc=True),
    scratch_types=[
        pltpu.VMEM((_C, _F), jnp.float32),
        pltpu.VMEM((_C, _F), jnp.float32),
        pltpu.VMEM((_C, _F), jnp.float32),
        pltpu.VMEM((_C, _Q), jnp.float32),
        pltpu.VMEM((_C, _Q), jnp.float32),
        pltpu.VMEM((_C, _Q), jnp.float32),
        pltpu.SemaphoreType.DMA,
        pltpu.SemaphoreType.DMA,
        pltpu.SemaphoreType.DMA,
        pltpu.SemaphoreType.DMA,
        pltpu.SemaphoreType.DMA,
        pltpu.SemaphoreType.DMA,
    ],
)
def _pool(x_hbm, out_hbm, in0, in1, in2, ou0, ou1, ou2,
          si0, si1, si2, so0, so1, so2):
    _pool_body(x_hbm, out_hbm, in0, in1, in2, ou0, ou1, ou2,
               si0, si1, si2, so0, so1, so2)


def kernel(x, groups):
    del groups
    xt = jnp.transpose(x, (0, 2, 1))
    out_t = _pool(xt)
    return jnp.transpose(out_t, (0, 2, 1))


# speedup vs baseline: 1.4837x; 1.0423x over previous
import functools

import jax
import jax.numpy as jnp
from jax import lax
from jax.experimental import pallas as pl
from jax.experimental.pallas import tpu as pltpu
from jax.experimental.pallas import tpu_sc as plsc

_BATCH = 4
_NPIX_FINE = 786432
_NPIX_COARSE = _NPIX_FINE // 4
_C = 16
_L = 16

_NC = 2
_NS = 16
_NW = _NC * _NS
_W_PER_B = _NW // _BATCH

_P_PER_W = _NPIX_FINE // _W_PER_B
_F = 3072
_Q = _F // 4
_NSTEPS = _P_PER_W // _F
_NBUF = 2
_NOUTER = _NSTEPS // _NBUF

_DNUMS = lax.GatherDimensionNumbers(
    offset_dims=(), collapsed_slice_dims=(0,), start_index_map=(0,)
)


def _dg(v, idx):
    return lax.gather(
        v, idx[:, None], _DNUMS, (1,),
        mode=lax.GatherScatterMode.PROMISE_IN_BOUNDS,
    )


def _pool_body(x_hbm, out_hbm, in0, in1, ou0, ou1, si0, si1, so0, so1):
    wid = lax.axis_index("s") * _NC + lax.axis_index("c")
    bi = lax.div(wid, _W_PER_B)
    rng = lax.rem(wid, _W_PER_B)
    p_base = rng * _P_PER_W
    q_base = rng * (_P_PER_W // 4)

    ins, outs = (in0, in1), (ou0, ou1)
    isems, osems = (si0, si1), (so0, so1)

    def in_copy(s, k):
        src = x_hbm.at[bi, :, pl.ds(p_base + s * _F, _F)]
        return pltpu.make_async_copy(src, ins[k], isems[k])

    def out_copy(s, k):
        dst = out_hbm.at[bi, :, pl.ds(q_base + s * _Q, _Q)]
        return pltpu.make_async_copy(outs[k], dst, osems[k])

    for k in range(_NBUF):
        in_copy(k, k).start()

    iot = lax.iota(jnp.int32, _L)
    xor1 = iot ^ 1
    xor2 = iot ^ 2
    trp = (iot & 3) * 4 + (iot >> 2)
    m4a = (iot & 3) == 0
    m4b = (iot & 3) == 1
    m4c = (iot & 3) == 2

    def outer(t, _):
        for k in range(_NBUF):
            s = t * _NBUF + k
            in_copy(s, k).wait()

            @plsc.parallel_loop(0, _Q // _L, unroll=4)
            def vbody(v, k=k):
                base = 64 * v
                for c in range(_C):
                    qs = []
                    for i in range(4):
                        vi = ins[k][c, pl.ds(base + _L * i, _L)]
                        t1 = vi + _dg(vi, xor1)
                        qs.append(t1 + _dg(t1, xor2))
                    s2 = jnp.where(
                        m4a, qs[0],
                        jnp.where(m4b, qs[1], jnp.where(m4c, qs[2], qs[3])),
                    )
                    outs[k][c, pl.ds(v * _L, _L)] = _dg(s2, trp) * 0.25

            @pl.when(s >= _NBUF)
            def _(k=k, s=s):
                out_copy(s - _NBUF, k).wait()

            out_copy(s, k).start()

            @pl.when(s + _NBUF < _NSTEPS)
            def _(k=k, s=s):
                in_copy(s + _NBUF, k).start()

        return 0

    lax.fori_loop(0, _NOUTER, outer, 0)

    for k in range(_NBUF):
        out_copy(_NSTEPS - _NBUF + k, k).wait()


@functools.partial(
    pl.kernel,
    out_type=jax.ShapeDtypeStruct((_BATCH, _C, _NPIX_COARSE), jnp.float32),
    mesh=plsc.VectorSubcoreMesh(core_axis_name="c", subcore_axis_name="s"),
    compiler_params=pltpu.CompilerParams(use_tc_tiling_on_sc=True),
    scratch_types=[
        pltpu.VMEM((_C, _F), jnp.float32),
        pltpu.VMEM((_C, _F), jnp.float32),
        pltpu.VMEM((_C, _Q), jnp.float32),
        pltpu.VMEM((_C, _Q), jnp.float32),
        pltpu.SemaphoreType.DMA,
        pltpu.SemaphoreType.DMA,
        pltpu.SemaphoreType.DMA,
        pltpu.SemaphoreType.DMA,
    ],
)
def _pool(x_hbm, out_hbm, in0, in1, ou0, ou1, si0, si1, so0, so1):
    _pool_body(x_hbm, out_hbm, in0, in1, ou0, ou1, si0, si1, so0, so1)


def kernel(x, groups):
    del groups
    xt = jnp.transpose(x, (0, 2, 1))
    out_t = _pool(xt)
    return jnp.transpose(out_t, (0, 2, 1))


# speedup vs baseline: 1.4996x; 1.0107x over previous
import functools

import jax
import jax.numpy as jnp
from jax import lax
from jax.experimental import pallas as pl
from jax.experimental.pallas import tpu as pltpu
from jax.experimental.pallas import tpu_sc as plsc

_BATCH = 4
_NPIX_FINE = 786432
_NPIX_COARSE = _NPIX_FINE // 4
_C = 16
_L = 16

_NC = 2
_NS = 16
_NW = _NC * _NS
_W_PER_B = _NW // _BATCH

_P_PER_W = _NPIX_FINE // _W_PER_B
_F = 2048
_Q = _F // 4
_NSTEPS = _P_PER_W // _F
_NBUF = 2
_NOUTER = _NSTEPS // _NBUF

_DNUMS = lax.GatherDimensionNumbers(
    offset_dims=(), collapsed_slice_dims=(0,), start_index_map=(0,)
)


def _dg(v, idx):
    return lax.gather(
        v, idx[:, None], _DNUMS, (1,),
        mode=lax.GatherScatterMode.PROMISE_IN_BOUNDS,
    )


def _pool_body(x_hbm, out_hbm, in0, in1, ou0, ou1, si0, si1, so0, so1):
    wid = lax.axis_index("s") * _NC + lax.axis_index("c")
    bi = lax.div(wid, _W_PER_B)
    rng = lax.rem(wid, _W_PER_B)
    p_base = rng * _P_PER_W
    q_base = rng * (_P_PER_W // 4)

    ins, outs = (in0, in1), (ou0, ou1)
    isems, osems = (si0, si1), (so0, so1)

    def in_copy(s, k):
        src = x_hbm.at[bi, :, pl.ds(p_base + s * _F, _F)]
        return pltpu.make_async_copy(src, ins[k], isems[k])

    def out_copy(s, k):
        dst = out_hbm.at[bi, :, pl.ds(q_base + s * _Q, _Q)]
        return pltpu.make_async_copy(outs[k], dst, osems[k])

    for k in range(_NBUF):
        in_copy(k, k).start()

    iot = lax.iota(jnp.int32, _L)
    xor1 = iot ^ 1
    xor2 = iot ^ 2
    trp = (iot & 3) * 4 + (iot >> 2)
    m4a = (iot & 3) == 0
    m4b = (iot & 3) == 1
    m4c = (iot & 3) == 2

    def outer(t, _):
        for k in range(_NBUF):
            s = t * _NBUF + k
            in_copy(s, k).wait()

            @plsc.parallel_loop(0, _Q // _L, unroll=4)
            def vbody(v, k=k):
                base = 64 * v
                for c in range(_C):
                    qs = []
                    for i in range(4):
                        vi = ins[k][c, pl.ds(base + _L * i, _L)]
                        t1 = vi + _dg(vi, xor1)
                        qs.append(t1 + _dg(t1, xor2))
                    s2 = jnp.where(
                        m4a, qs[0],
                        jnp.where(m4b, qs[1], jnp.where(m4c, qs[2], qs[3])),
                    )
                    outs[k][c, pl.ds(v * _L, _L)] = _dg(s2, trp) * 0.25

            @pl.when(s >= _NBUF)
            def _(k=k, s=s):
                out_copy(s - _NBUF, k).wait()

            out_copy(s, k).start()

            @pl.when(s + _NBUF < _NSTEPS)
            def _(k=k, s=s):
                in_copy(s + _NBUF, k).start()

        return 0

    lax.fori_loop(0, _NOUTER, outer, 0)

    for k in range(_NBUF):
        out_copy(_NSTEPS - _NBUF + k, k).wait()


@functools.partial(
    pl.kernel,
    out_type=jax.ShapeDtypeStruct((_BATCH, _C, _NPIX_COARSE), jnp.float32),
    mesh=plsc.VectorSubcoreMesh(core_axis_name="c", subcore_axis_name="s"),
    compiler_params=pltpu.CompilerParams(use_tc_tiling_on_sc=True),
    scratch_types=[
        pltpu.VMEM((_C, _F), jnp.float32),
        pltpu.VMEM((_C, _F), jnp.float32),
        pltpu.VMEM((_C, _Q), jnp.float32),
        pltpu.VMEM((_C, _Q), jnp.float32),
        pltpu.SemaphoreType.DMA,
        pltpu.SemaphoreType.DMA,
        pltpu.SemaphoreType.DMA,
        pltpu.SemaphoreType.DMA,
    ],
)
def _pool(x_hbm, out_hbm, in0, in1, ou0, ou1, si0, si1, so0, so1):
    _pool_body(x_hbm, out_hbm, in0, in1, ou0, ou1, si0, si1, so0, so1)


def kernel(x, groups):
    del groups
    xt = jnp.transpose(x, (0, 2, 1))
    out_t = _pool(xt)
    return jnp.transpose(out_t, (0, 2, 1))
